# baseline (speedup 1.0000x reference)
import jax
import jax.numpy as jnp
from jax.experimental import pallas as pl
from jax.experimental.pallas import tpu as pltpu

K = 16384
DIM = 2048
N_CLASS = 2
KNN = 8
ALPHA = 0.5

BK = 1024
NB = K // BK
NT = DIM // 128


def _fused_kernel(q_ref, x_ref, lab_ref, labc_ref, labf_ref, pred_ref, score_ref,
                  kl0_ref, kl1_ref, stot_ref, s1_ref, momy_ref):
    pid = pl.program_id(0)
    x = x_ref[...]
    q = q_ref[...]
    lab = lab_ref[0]
    maskc = (labc_ref[0] == 1).astype(jnp.float32)

    inv_d = jnp.float32(1.0 / DIM)

    @pl.when(pid == 0)
    def _init():
        stot_ref[...] = jnp.zeros((8, DIM), jnp.float32)
        s1_ref[...] = jnp.zeros((8, DIM), jnp.float32)
        mq = jnp.sum(q) * inv_d
        momy_ref[...] = (q - mq) ** 2

    mom_y = momy_ref[...]

    xs = [x[:, c * 128:(c + 1) * 128] for c in range(NT)]
    ms = [mom_y[:, c * 128:(c + 1) * 128] for c in range(NT)]
    rs128 = xs[0]
    for c in range(1, NT):
        rs128 = rs128 + xs[c]
    for c in range(NT):
        sl = slice(c * 128, (c + 1) * 128)
        stot_ref[:, sl] += jnp.sum(
            xs[c].reshape(BK // 8, 8, 128), axis=0)
        s1_ref[:, sl] += jnp.sum(
            (xs[c] * maskc).reshape(BK // 8, 8, 128), axis=0)
    ones_col = jnp.ones((128, 1), jnp.float32)
    rs = jnp.dot(rs128, ones_col,
                 preferred_element_type=jnp.float32) * inv_d

    acc = None
    for c in range(NT):
        dc = xs[c] - rs
        tc = jnp.abs(dc * dc - ms[c])
        acc = tc if acc is None else acc + tc
    ones_row = jnp.ones((1, 128), jnp.float32)
    kl_row = jax.lax.dot_general(
        ones_row, acc, (((1,), (1,)), ((), ())),
        preferred_element_type=jnp.float32)

    inf = jnp.float32(jnp.inf)
    kl0_ref[pl.ds(pid, 1), :] = jnp.where(lab == 0, kl_row, inf)
    kl1_ref[pl.ds(pid, 1), :] = jnp.where(lab == 1, kl_row, inf)

    @pl.when(pid == NB - 1)
    def _epilogue():
        c1 = jnp.sum((labf_ref[...] == 1).astype(jnp.float32))
        c0 = jnp.float32(K) - c1
        s1 = jnp.sum(s1_ref[...], axis=0, keepdims=True)
        s0 = jnp.sum(stot_ref[...], axis=0, keepdims=True) - s1
        ctr0 = s0 / c0
        ctr1 = s1 / c1
        logq = jnp.log(q)
        ccd0 = jnp.sum(ctr0 * (jnp.log(ctr0) - logq)) * inv_d
        ccd1 = jnp.sum(ctr1 * (jnp.log(ctr1) - logq)) * inv_d

        ii = (jax.lax.broadcasted_iota(jnp.int32, (NB, BK), 0) * BK
              + jax.lax.broadcasted_iota(jnp.int32, (NB, BK), 1))

        def top8_pair(va, vb):
            outs_a, outs_b = [], []
            for _ in range(KNN):
                ma = jnp.min(va)
                mb = jnp.min(vb)
                fa = jnp.min(jnp.where(va == ma, ii, K))
                fb = jnp.min(jnp.where(vb == mb, ii, K))
                va = jnp.where(ii == fa, inf, va)
                vb = jnp.where(ii == fb, inf, vb)
                outs_a.append(ma)
                outs_b.append(mb)
            return outs_a, outs_b

        v0, v1 = top8_pair(kl0_ref[...], kl1_ref[...])
        inst0 = sum(v0) * (1.0 / KNN)
        inst1 = sum(v1) * (1.0 / KNN)

        sc0 = ccd0 * ALPHA + inst0 * (1.0 - ALPHA)
        sc1 = ccd1 * ALPHA + inst1 * (1.0 - ALPHA)
        nrm = jnp.maximum(jnp.abs(sc0) + jnp.abs(sc1), 1e-12)

        m0 = [v * (1.0 - ALPHA) + ccd0 * ALPHA for v in v0]
        m1 = [v * (1.0 - ALPHA) + ccd1 * ALPHA for v in v1]
        count0 = jnp.int32(0)
        for i in range(KNN):
            count0 += (m0[i] < m1[KNN - 1 - i]).astype(jnp.int32)
        pred_val = jnp.where(count0 * 2 >= KNN, 0, 1).astype(jnp.int32)

        lane = jax.lax.broadcasted_iota(jnp.int32, (1, 128), 1)
        score_ref[...] = jnp.where(
            lane == 0, sc0 / nrm, jnp.where(lane == 1, sc1 / nrm, 0.0)
        ).astype(jnp.float32)
        pred_ref[...] = jnp.zeros((1, 128), jnp.int32) + pred_val


def kernel(query, queue_anchor, queue_label):
    labels3 = queue_label.reshape(NB, 1, BK)
    pred2, score2 = pl.pallas_call(
        _fused_kernel,
        grid=(NB,),
        in_specs=[
            pl.BlockSpec((1, DIM), lambda i: (0, 0)),
            pl.BlockSpec((BK, DIM), lambda i: (i, 0)),
            pl.BlockSpec((1, 1, BK), lambda i: (i, 0, 0)),
            pl.BlockSpec((1, BK, 1), lambda i: (i, 0, 0)),
            pl.BlockSpec((NB, 1, BK), lambda i: (0, 0, 0)),
        ],
        out_specs=[
            pl.BlockSpec((1, 128), lambda i: (0, 0)),
            pl.BlockSpec((1, 128), lambda i: (0, 0)),
        ],
        out_shape=[
            jax.ShapeDtypeStruct((1, 128), jnp.int32),
            jax.ShapeDtypeStruct((1, 128), jnp.float32),
        ],
        scratch_shapes=[
            pltpu.VMEM((NB, BK), jnp.float32),
            pltpu.VMEM((NB, BK), jnp.float32),
            pltpu.VMEM((8, DIM), jnp.float32),
            pltpu.VMEM((8, DIM), jnp.float32),
            pltpu.VMEM((1, DIM), jnp.float32),
        ],
    )(query, queue_anchor, labels3, labels3.reshape(NB, BK, 1), labels3)
    return pred2[0, :1], score2[0, :2]


# speedup vs baseline: 1.3116x; 1.3116x over previous
import jax
import jax.numpy as jnp
from jax.experimental import pallas as pl
from jax.experimental.pallas import tpu as pltpu

K = 16384
DIM = 2048
N_CLASS = 2
KNN = 8
ALPHA = 0.5

BK = 1024
NB = K // BK
NT = DIM // 128


def _fused_kernel(q_ref, x_ref, lab_ref, labf_ref, pred_ref, score_ref,
                  kl0_ref, kl1_ref, s_ref, momy_ref):
    pid = pl.program_id(0)
    x = x_ref[...]
    q = q_ref[...]
    lab = lab_ref[0]
    mask1 = (lab == 1).astype(jnp.float32)

    w = jnp.concatenate([jnp.ones((1, BK), jnp.float32), mask1], axis=0)
    part = jnp.dot(w, x, preferred_element_type=jnp.float32)

    inv_d = jnp.float32(1.0 / DIM)

    @pl.when(pid == 0)
    def _init():
        s_ref[...] = part
        mq = jnp.sum(q) * inv_d
        momy_ref[...] = (q - mq) ** 2

    @pl.when(pid != 0)
    def _acc():
        s_ref[...] += part

    mom_y = momy_ref[...]

    xs = [x[:, c * 128:(c + 1) * 128] for c in range(NT)]
    ms = [mom_y[:, c * 128:(c + 1) * 128] for c in range(NT)]
    rs128 = xs[0]
    for c in range(1, NT):
        rs128 = rs128 + xs[c]
    ones_col = jnp.ones((128, 1), jnp.float32)
    rs = jnp.dot(rs128, ones_col,
                 preferred_element_type=jnp.float32) * inv_d

    acc = None
    for c in range(NT):
        dc = xs[c] - rs
        tc = jnp.abs(dc * dc - ms[c])
        acc = tc if acc is None else acc + tc
    ones_row = jnp.ones((1, 128), jnp.float32)
    kl_row = jax.lax.dot_general(
        ones_row, acc, (((1,), (1,)), ((), ())),
        preferred_element_type=jnp.float32)

    inf = jnp.float32(jnp.inf)
    kl0_ref[pl.ds(pid, 1), :] = jnp.where(lab == 0, kl_row, inf)
    kl1_ref[pl.ds(pid, 1), :] = jnp.where(lab == 1, kl_row, inf)

    @pl.when(pid == NB - 1)
    def _epilogue():
        c1 = jnp.sum((labf_ref[...] == 1).astype(jnp.float32))
        c0 = jnp.float32(K) - c1
        s0 = s_ref[0:1, :] - s_ref[1:2, :]
        s1 = s_ref[1:2, :]
        ctr0 = s0 / c0
        ctr1 = s1 / c1
        logq = jnp.log(q)
        ccd0 = jnp.sum(ctr0 * (jnp.log(ctr0) - logq)) * inv_d
        ccd1 = jnp.sum(ctr1 * (jnp.log(ctr1) - logq)) * inv_d

        ii = (jax.lax.broadcasted_iota(jnp.int32, (NB, BK), 0) * BK
              + jax.lax.broadcasted_iota(jnp.int32, (NB, BK), 1))

        def top8_pair(va, vb):
            outs_a, outs_b = [], []
            for _ in range(KNN):
                ma = jnp.min(va)
                mb = jnp.min(vb)
                fa = jnp.min(jnp.where(va == ma, ii, K))
                fb = jnp.min(jnp.where(vb == mb, ii, K))
                va = jnp.where(ii == fa, inf, va)
                vb = jnp.where(ii == fb, inf, vb)
                outs_a.append(ma)
                outs_b.append(mb)
            return outs_a, outs_b

        v0, v1 = top8_pair(kl0_ref[...], kl1_ref[...])
        inst0 = sum(v0) * (1.0 / KNN)
        inst1 = sum(v1) * (1.0 / KNN)

        sc0 = ccd0 * ALPHA + inst0 * (1.0 - ALPHA)
        sc1 = ccd1 * ALPHA + inst1 * (1.0 - ALPHA)
        nrm = jnp.maximum(jnp.abs(sc0) + jnp.abs(sc1), 1e-12)

        m0 = [v * (1.0 - ALPHA) + ccd0 * ALPHA for v in v0]
        m1 = [v * (1.0 - ALPHA) + ccd1 * ALPHA for v in v1]
        count0 = jnp.int32(0)
        for i in range(KNN):
            count0 += (m0[i] < m1[KNN - 1 - i]).astype(jnp.int32)
        pred_val = jnp.where(count0 * 2 >= KNN, 0, 1).astype(jnp.int32)

        lane = jax.lax.broadcasted_iota(jnp.int32, (1, 128), 1)
        score_ref[...] = jnp.where(
            lane == 0, sc0 / nrm, jnp.where(lane == 1, sc1 / nrm, 0.0)
        ).astype(jnp.float32)
        pred_ref[...] = jnp.zeros((1, 128), jnp.int32) + pred_val


def kernel(query, queue_anchor, queue_label):
    labels3 = queue_label.reshape(NB, 1, BK)
    pred2, score2 = pl.pallas_call(
        _fused_kernel,
        grid=(NB,),
        in_specs=[
            pl.BlockSpec((1, DIM), lambda i: (0, 0)),
            pl.BlockSpec((BK, DIM), lambda i: (i, 0)),
            pl.BlockSpec((1, 1, BK), lambda i: (i, 0, 0)),
            pl.BlockSpec((NB, 1, BK), lambda i: (0, 0, 0)),
        ],
        out_specs=[
            pl.BlockSpec((1, 128), lambda i: (0, 0)),
            pl.BlockSpec((1, 128), lambda i: (0, 0)),
        ],
        out_shape=[
            jax.ShapeDtypeStruct((1, 128), jnp.int32),
            jax.ShapeDtypeStruct((1, 128), jnp.float32),
        ],
        scratch_shapes=[
            pltpu.VMEM((NB, BK), jnp.float32),
            pltpu.VMEM((NB, BK), jnp.float32),
            pltpu.VMEM((2, DIM), jnp.float32),
            pltpu.VMEM((1, DIM), jnp.float32),
        ],
    )(query, queue_anchor, labels3, labels3)
    return pred2[0, :1], score2[0, :2]
